# baseline (speedup 1.0000x reference)
import functools

import jax
import jax.numpy as jnp
from jax import lax
from jax.experimental import pallas as pl
from jax.experimental.pallas import tpu as pltpu
from jax.experimental.pallas import tpu_sc as plsc

NUM_CORES = 2
NUM_SUBCORES = 16
NW = NUM_CORES * NUM_SUBCORES


def _pack_bf16_pairs(x):
    k = x.shape[-1] // 2
    a = x[..., :k].astype(jnp.bfloat16).astype(jnp.float32)
    b = x[..., k:].astype(jnp.bfloat16).astype(jnp.float32)
    ua = lax.shift_right_logical(lax.bitcast_convert_type(a, jnp.uint32),
                                 jnp.uint32(16))
    ub = lax.bitcast_convert_type(b, jnp.uint32) & jnp.uint32(0xFFFF0000)
    return lax.bitcast_convert_type(ua | ub, jnp.float32)


def _unpack_bf16_pairs(p):
    u = lax.bitcast_convert_type(p, jnp.uint32)
    a = lax.bitcast_convert_type(lax.shift_left(u, jnp.uint32(16)), jnp.float32)
    b = lax.bitcast_convert_type(u & jnp.uint32(0xFFFF0000), jnp.float32)
    return jnp.concatenate([a, b], axis=-1)


def _silu(x):
    return x * jax.nn.sigmoid(x)


def _layer_norm(x, g, b, eps=1e-5):
    m = jnp.mean(x, axis=-1, keepdims=True)
    v = jnp.var(x, axis=-1, keepdims=True)
    return (x - m) / jnp.sqrt(v + eps) * g + b


def _node_proj(mesh_nfeat, grid_nfeat, Ws, Wd, b1, blk):
    n, d = mesh_nfeat.shape

    def body(mesh_ref, grid_ref, ws_ref, wd_ref, b1_ref, xs_ref, xd_ref):
        xs_ref[...] = _pack_bf16_pairs(lax.dot_general(
            mesh_ref[...], ws_ref[...], (((1,), (1,)), ((), ())),
            preferred_element_type=jnp.float32))
        xd_ref[...] = _pack_bf16_pairs(lax.dot_general(
            grid_ref[...], wd_ref[...], (((1,), (1,)), ((), ())),
            preferred_element_type=jnp.float32) + b1_ref[...])

    return pl.pallas_call(
        body,
        grid=(n // blk,),
        in_specs=[
            pl.BlockSpec((blk, d), lambda i: (i, 0)),
            pl.BlockSpec((blk, d), lambda i: (i, 0)),
            pl.BlockSpec(Ws.shape, lambda i: (0, 0)),
            pl.BlockSpec(Wd.shape, lambda i: (0, 0)),
            pl.BlockSpec((1, d), lambda i: (0, 0)),
        ],
        out_specs=[
            pl.BlockSpec((blk, d // 2), lambda i: (i, 0)),
            pl.BlockSpec((blk, d // 2), lambda i: (i, 0)),
        ],
        out_shape=[
            jax.ShapeDtypeStruct((n, d // 2), jnp.float32),
            jax.ShapeDtypeStruct((n, d // 2), jnp.float32),
        ],
    )(mesh_nfeat, grid_nfeat, Ws, Wd, b1)


def _gather_pair(xs, xd, src, dst, chunk):
    e = src.shape[0]
    d = xs.shape[1]
    per_w = e // NW
    n_chunks = per_w // chunk
    mesh = plsc.VectorSubcoreMesh(
        core_axis_name="c", subcore_axis_name="s",
        num_cores=NUM_CORES, num_subcores=NUM_SUBCORES)

    @functools.partial(
        pl.kernel,
        out_type=jax.ShapeDtypeStruct((e, 2 * d), jnp.float32),
        mesh=mesh,
        compiler_params=pltpu.CompilerParams(use_tc_tiling_on_sc=False),
        scratch_types=[
            pltpu.VMEM((chunk,), jnp.int32),
            pltpu.VMEM((chunk,), jnp.int32),
            pltpu.VMEM((chunk, d), jnp.float32),
            pltpu.VMEM((chunk, d), jnp.float32),
            pltpu.SemaphoreType.DMA,
            pltpu.SemaphoreType.DMA,
        ],
    )
    def k(xs_hbm, xd_hbm, src_hbm, dst_hbm, g_hbm,
          idx_s, idx_d, bufa, bufb, sem1, sem2):
        wid = lax.axis_index("s") * NUM_CORES + lax.axis_index("c")

        def chunk_body(c, carry):
            base = wid * per_w + c * chunk
            pltpu.sync_copy(src_hbm.at[pl.ds(base, chunk)], idx_s)
            pltpu.sync_copy(dst_hbm.at[pl.ds(base, chunk)], idx_d)
            cp1 = pltpu.async_copy(xs_hbm.at[idx_s], bufa, sem1)
            cp2 = pltpu.async_copy(xd_hbm.at[idx_d], bufb, sem2)
            cp1.wait()
            cp2.wait()
            pltpu.sync_copy(bufa, g_hbm.at[pl.ds(base, chunk), pl.ds(0, d)])
            pltpu.sync_copy(bufb, g_hbm.at[pl.ds(base, chunk), pl.ds(d, d)])
            return carry

        lax.fori_loop(0, n_chunks, chunk_body, 0)

    return k(xs, xd, src, dst)


def _edge_mlp(m2g, g, We, W2, b2, ln_g, ln_b, blk):
    e, d = m2g.shape

    def body(m2g_ref, g_ref, we_ref, w2_ref, b2_ref, lg_ref, lb_ref,
             out_ref):
        h = lax.dot_general(
            m2g_ref[...], we_ref[...], (((1,), (1,)), ((), ())),
            preferred_element_type=jnp.float32)
        gblk = g_ref[...]
        h = (h + _unpack_bf16_pairs(gblk[:, :d // 2])
             + _unpack_bf16_pairs(gblk[:, d // 2:]))
        h = _silu(h)
        p = lax.dot_general(
            h, w2_ref[...], (((1,), (1,)), ((), ())),
            preferred_element_type=jnp.float32) + b2_ref[...]
        out_ref[...] = _layer_norm(p, lg_ref[...], lb_ref[...])

    return pl.pallas_call(
        body,
        grid=(e // blk,),
        in_specs=[
            pl.BlockSpec((blk, d), lambda i: (i, 0)),
            pl.BlockSpec((blk, d), lambda i: (i, 0)),
            pl.BlockSpec(We.shape, lambda i: (0, 0)),
            pl.BlockSpec(W2.shape, lambda i: (0, 0)),
            pl.BlockSpec((1, d), lambda i: (0, 0)),
            pl.BlockSpec((1, d), lambda i: (0, 0)),
            pl.BlockSpec((1, d), lambda i: (0, 0)),
        ],
        out_specs=pl.BlockSpec((blk, d), lambda i: (i, 0)),
        out_shape=jax.ShapeDtypeStruct((e, d), jnp.float32),
        compiler_params=pltpu.CompilerParams(
            dimension_semantics=("parallel",)),
    )(m2g, g, We, W2, b2, ln_g, ln_b)


def _scatter_sum(efeat, dst, n_nodes, chunk):
    e, d = efeat.shape
    per_w = e // NW
    n_chunks = per_w // chunk
    base_rows = (n_nodes // NUM_SUBCORES) // 8 * 8
    last_rows = n_nodes - base_rows * (NUM_SUBCORES - 1)
    mesh = plsc.VectorSubcoreMesh(
        core_axis_name="c", subcore_axis_name="s",
        num_cores=NUM_CORES, num_subcores=NUM_SUBCORES)

    @functools.partial(
        pl.kernel,
        out_type=jax.ShapeDtypeStruct((NUM_CORES, n_nodes, d), jnp.float32),
        mesh=mesh,
        scratch_types=[
            pltpu.VMEM((chunk,), jnp.int32),
            pltpu.VMEM((chunk, d), jnp.float32),
            pltpu.VMEM_SHARED((n_nodes, d), jnp.float32),
            pltpu.SemaphoreType.DMA,
        ],
    )
    def k(ef_hbm, dst_hbm, out_hbm, idx, rows, acc, sem):
        cid = lax.axis_index("c")
        sid = lax.axis_index("s")
        wid = sid * NUM_CORES + cid

        def zero_row(r, carry):
            for j in range(d // 16):
                rows[r, pl.ds(j * 16, 16)] = jnp.zeros((16,), jnp.float32)
            return carry

        lax.fori_loop(0, chunk, zero_row, 0)
        done = 0
        while done < base_rows:
            step = min(chunk, base_rows - done)
            pltpu.sync_copy(rows.at[pl.ds(0, step)],
                            acc.at[pl.ds(sid * base_rows + done, step)])
            done += step

        extra = last_rows - base_rows

        @pl.when(sid == NUM_SUBCORES - 1)
        def _zero_tail():
            pltpu.sync_copy(
                rows.at[pl.ds(0, extra)],
                acc.at[pl.ds(base_rows * NUM_SUBCORES, extra)])

        plsc.subcore_barrier()

        def chunk_body(c, carry):
            base = wid * per_w + c * chunk
            pltpu.sync_copy(dst_hbm.at[pl.ds(base, chunk)], idx)
            pltpu.sync_copy(ef_hbm.at[pl.ds(base, chunk)], rows)
            pltpu.sync_copy(rows, acc.at[idx], add=True)
            return carry

        lax.fori_loop(0, n_chunks, chunk_body, 0)
        plsc.subcore_barrier()
        pltpu.sync_copy(acc.at[pl.ds(sid * base_rows, base_rows)],
                        out_hbm.at[cid, pl.ds(sid * base_rows, base_rows)])

        @pl.when(sid == NUM_SUBCORES - 1)
        def _write_tail():
            pltpu.sync_copy(
                acc.at[pl.ds(base_rows * NUM_SUBCORES, extra)],
                out_hbm.at[cid, pl.ds(base_rows * NUM_SUBCORES, extra)])

    return k(efeat, dst)


def _node_mlp(parts, grid_nfeat, Wn1a, Wn1b, bn1, Wn2, bn2, ln_g, ln_b, blk):
    n, d = grid_nfeat.shape

    def body(parts_ref, grid_ref, w1a_ref, w1b_ref, b1_ref, w2_ref, b2_ref,
             lg_ref, lb_ref, out_ref):
        agg = parts_ref[0] + parts_ref[1]
        grid_blk = grid_ref[...]
        h = lax.dot_general(
            agg, w1a_ref[...], (((1,), (1,)), ((), ())),
            preferred_element_type=jnp.float32)
        h = h + lax.dot_general(
            grid_blk, w1b_ref[...], (((1,), (1,)), ((), ())),
            preferred_element_type=jnp.float32) + b1_ref[...]
        h = _silu(h)
        p = lax.dot_general(
            h, w2_ref[...], (((1,), (1,)), ((), ())),
            preferred_element_type=jnp.float32) + b2_ref[...]
        out_ref[...] = _layer_norm(p, lg_ref[...], lb_ref[...]) + grid_blk

    return pl.pallas_call(
        body,
        grid=(n // blk,),
        in_specs=[
            pl.BlockSpec((NUM_CORES, blk, d), lambda i: (0, i, 0)),
            pl.BlockSpec((blk, d), lambda i: (i, 0)),
            pl.BlockSpec(Wn1a.shape, lambda i: (0, 0)),
            pl.BlockSpec(Wn1b.shape, lambda i: (0, 0)),
            pl.BlockSpec((1, d), lambda i: (0, 0)),
            pl.BlockSpec(Wn2.shape, lambda i: (0, 0)),
            pl.BlockSpec((1, d), lambda i: (0, 0)),
            pl.BlockSpec((1, d), lambda i: (0, 0)),
            pl.BlockSpec((1, d), lambda i: (0, 0)),
        ],
        out_specs=pl.BlockSpec((blk, d), lambda i: (i, 0)),
        out_shape=jax.ShapeDtypeStruct((n, d), jnp.float32),
    )(parts, grid_nfeat, Wn1a, Wn1b, bn1, Wn2, bn2, ln_g, ln_b)


def kernel(m2g_efeat, grid_nfeat, mesh_nfeat, edge_index,
           We, Ws, Wd, b1, W2, b2, ln_e_g, ln_e_b,
           Wn1, bn1, Wn2, bn2, ln_n_g, ln_n_b):
    e, d = m2g_efeat.shape
    n = grid_nfeat.shape[0]
    src = edge_index[0].astype(jnp.int32)
    dst = edge_index[1].astype(jnp.int32)

    row = lambda v: v.reshape(1, -1)

    xs, xd = _node_proj(mesh_nfeat, grid_nfeat, Ws, Wd, row(b1), blk=1000)
    g = _gather_pair(xs, xd, src, dst, chunk=400)
    efeat = _edge_mlp(m2g_efeat, g, We, W2, row(b2), row(ln_e_g),
                      row(ln_e_b), blk=1000)
    parts = _scatter_sum(efeat, dst, n, chunk=200)
    out = _node_mlp(parts, grid_nfeat, Wn1[:, :d], Wn1[:, d:], row(bn1),
                    Wn2, row(bn2), row(ln_n_g), row(ln_n_b), blk=1000)
    return out


# speedup vs baseline: 1.1275x; 1.1275x over previous
import functools

import jax
import jax.numpy as jnp
from jax import lax
from jax.experimental import pallas as pl
from jax.experimental.pallas import tpu as pltpu
from jax.experimental.pallas import tpu_sc as plsc

NUM_CORES = 2
NUM_SUBCORES = 16
NW = NUM_CORES * NUM_SUBCORES


def _pack_bf16_pairs(x):
    k = x.shape[-1] // 2
    a = x[..., :k].astype(jnp.bfloat16).astype(jnp.float32)
    b = x[..., k:].astype(jnp.bfloat16).astype(jnp.float32)
    ua = lax.shift_right_logical(lax.bitcast_convert_type(a, jnp.uint32),
                                 jnp.uint32(16))
    ub = lax.bitcast_convert_type(b, jnp.uint32) & jnp.uint32(0xFFFF0000)
    return lax.bitcast_convert_type(ua | ub, jnp.float32)


def _unpack_bf16_pairs(p):
    u = lax.bitcast_convert_type(p, jnp.uint32)
    a = lax.bitcast_convert_type(lax.shift_left(u, jnp.uint32(16)), jnp.float32)
    b = lax.bitcast_convert_type(u & jnp.uint32(0xFFFF0000), jnp.float32)
    return jnp.concatenate([a, b], axis=-1)


def _silu(x):
    return x * jax.nn.sigmoid(x)


def _layer_norm(x, g, b, eps=1e-5):
    m = jnp.mean(x, axis=-1, keepdims=True)
    v = jnp.var(x, axis=-1, keepdims=True)
    return (x - m) / jnp.sqrt(v + eps) * g + b


def _node_proj(mesh_nfeat, grid_nfeat, Ws, Wd, b1, blk):
    n, d = mesh_nfeat.shape

    def body(mesh_ref, grid_ref, ws_ref, wd_ref, b1_ref, xs_ref, xd_ref):
        xs_ref[...] = _pack_bf16_pairs(lax.dot_general(
            mesh_ref[...], ws_ref[...], (((1,), (1,)), ((), ())),
            preferred_element_type=jnp.float32))
        xd_ref[...] = _pack_bf16_pairs(lax.dot_general(
            grid_ref[...], wd_ref[...], (((1,), (1,)), ((), ())),
            preferred_element_type=jnp.float32) + b1_ref[...])

    return pl.pallas_call(
        body,
        grid=(n // blk,),
        in_specs=[
            pl.BlockSpec((blk, d), lambda i: (i, 0)),
            pl.BlockSpec((blk, d), lambda i: (i, 0)),
            pl.BlockSpec(Ws.shape, lambda i: (0, 0)),
            pl.BlockSpec(Wd.shape, lambda i: (0, 0)),
            pl.BlockSpec((1, d), lambda i: (0, 0)),
        ],
        out_specs=[
            pl.BlockSpec((blk, d // 2), lambda i: (i, 0)),
            pl.BlockSpec((blk, d // 2), lambda i: (i, 0)),
        ],
        out_shape=[
            jax.ShapeDtypeStruct((n, d // 2), jnp.float32),
            jax.ShapeDtypeStruct((n, d // 2), jnp.float32),
        ],
    )(mesh_nfeat, grid_nfeat, Ws, Wd, b1)


def _gather_pair(xs, xd, src, dst, chunk):
    e = src.shape[0]
    d = xs.shape[1]
    per_w = e // NW
    n_chunks = per_w // chunk
    mesh = plsc.VectorSubcoreMesh(
        core_axis_name="c", subcore_axis_name="s",
        num_cores=NUM_CORES, num_subcores=NUM_SUBCORES)

    @functools.partial(
        pl.kernel,
        out_type=jax.ShapeDtypeStruct((e, 2 * d), jnp.float32),
        mesh=mesh,
        compiler_params=pltpu.CompilerParams(use_tc_tiling_on_sc=False),
        scratch_types=[
            pltpu.VMEM((chunk,), jnp.int32),
            pltpu.VMEM((chunk,), jnp.int32),
            pltpu.VMEM((chunk, d), jnp.float32),
            pltpu.VMEM((chunk, d), jnp.float32),
            pltpu.SemaphoreType.DMA,
            pltpu.SemaphoreType.DMA,
        ],
    )
    def k(xs_hbm, xd_hbm, src_hbm, dst_hbm, g_hbm,
          idx_s, idx_d, bufa, bufb, sem1, sem2):
        wid = lax.axis_index("s") * NUM_CORES + lax.axis_index("c")

        def chunk_body(c, carry):
            base = wid * per_w + c * chunk
            pltpu.sync_copy(src_hbm.at[pl.ds(base, chunk)], idx_s)
            pltpu.sync_copy(dst_hbm.at[pl.ds(base, chunk)], idx_d)
            cp1 = pltpu.async_copy(xs_hbm.at[idx_s], bufa, sem1)
            cp2 = pltpu.async_copy(xd_hbm.at[idx_d], bufb, sem2)
            cp1.wait()
            cp2.wait()
            pltpu.sync_copy(bufa, g_hbm.at[pl.ds(base, chunk), pl.ds(0, d)])
            pltpu.sync_copy(bufb, g_hbm.at[pl.ds(base, chunk), pl.ds(d, d)])
            return carry

        lax.fori_loop(0, n_chunks, chunk_body, 0)

    return k(xs, xd, src, dst)


def _edge_mlp(m2g, g, We, W2, b2, ln_g, ln_b, blk):
    e, d = m2g.shape

    def body(m2g_ref, g_ref, we_ref, w2_ref, b2_ref, lg_ref, lb_ref,
             out_ref):
        h = lax.dot_general(
            m2g_ref[...], we_ref[...], (((1,), (1,)), ((), ())),
            preferred_element_type=jnp.float32)
        gblk = g_ref[...]
        h = (h + _unpack_bf16_pairs(gblk[:, :d // 2])
             + _unpack_bf16_pairs(gblk[:, d // 2:]))
        h = _silu(h)
        p = lax.dot_general(
            h, w2_ref[...], (((1,), (1,)), ((), ())),
            preferred_element_type=jnp.float32) + b2_ref[...]
        out_ref[...] = _layer_norm(p, lg_ref[...], lb_ref[...])

    return pl.pallas_call(
        body,
        grid=(e // blk,),
        in_specs=[
            pl.BlockSpec((blk, d), lambda i: (i, 0)),
            pl.BlockSpec((blk, d), lambda i: (i, 0)),
            pl.BlockSpec(We.shape, lambda i: (0, 0)),
            pl.BlockSpec(W2.shape, lambda i: (0, 0)),
            pl.BlockSpec((1, d), lambda i: (0, 0)),
            pl.BlockSpec((1, d), lambda i: (0, 0)),
            pl.BlockSpec((1, d), lambda i: (0, 0)),
        ],
        out_specs=pl.BlockSpec((blk, d), lambda i: (i, 0)),
        out_shape=jax.ShapeDtypeStruct((e, d), jnp.float32),
        compiler_params=pltpu.CompilerParams(
            dimension_semantics=("parallel",)),
    )(m2g, g, We, W2, b2, ln_g, ln_b)


def _scatter_sum(efeat, dst, n_nodes, chunk):
    e, d = efeat.shape
    per_w = e // NW
    n_chunks = per_w // chunk
    base_rows = (n_nodes // NUM_SUBCORES) // 8 * 8
    last_rows = n_nodes - base_rows * (NUM_SUBCORES - 1)
    mesh = plsc.VectorSubcoreMesh(
        core_axis_name="c", subcore_axis_name="s",
        num_cores=NUM_CORES, num_subcores=NUM_SUBCORES)

    @functools.partial(
        pl.kernel,
        out_type=jax.ShapeDtypeStruct((NUM_CORES, n_nodes, d), jnp.float32),
        mesh=mesh,
        scratch_types=[
            pltpu.VMEM((chunk,), jnp.int32),
            pltpu.VMEM((chunk, d), jnp.float32),
            pltpu.VMEM_SHARED((n_nodes, d), jnp.float32),
            pltpu.SemaphoreType.DMA,
        ],
    )
    def k(ef_hbm, dst_hbm, out_hbm, idx, rows, acc, sem):
        cid = lax.axis_index("c")
        sid = lax.axis_index("s")
        wid = sid * NUM_CORES + cid

        def zero_row(r, carry):
            for j in range(d // 16):
                rows[r, pl.ds(j * 16, 16)] = jnp.zeros((16,), jnp.float32)
            return carry

        lax.fori_loop(0, chunk, zero_row, 0)
        done = 0
        while done < base_rows:
            step = min(chunk, base_rows - done)
            pltpu.sync_copy(rows.at[pl.ds(0, step)],
                            acc.at[pl.ds(sid * base_rows + done, step)])
            done += step

        extra = last_rows - base_rows

        @pl.when(sid == NUM_SUBCORES - 1)
        def _zero_tail():
            pltpu.sync_copy(
                rows.at[pl.ds(0, extra)],
                acc.at[pl.ds(base_rows * NUM_SUBCORES, extra)])

        plsc.subcore_barrier()

        def chunk_body(c, carry):
            base = wid * per_w + c * chunk
            pltpu.sync_copy(dst_hbm.at[pl.ds(base, chunk)], idx)
            pltpu.sync_copy(ef_hbm.at[pl.ds(base, chunk)], rows)
            pltpu.sync_copy(rows, acc.at[idx], add=True)
            return carry

        lax.fori_loop(0, n_chunks, chunk_body, 0)
        plsc.subcore_barrier()
        pltpu.sync_copy(acc.at[pl.ds(sid * base_rows, base_rows)],
                        out_hbm.at[cid, pl.ds(sid * base_rows, base_rows)])

        @pl.when(sid == NUM_SUBCORES - 1)
        def _write_tail():
            pltpu.sync_copy(
                acc.at[pl.ds(base_rows * NUM_SUBCORES, extra)],
                out_hbm.at[cid, pl.ds(base_rows * NUM_SUBCORES, extra)])

    return k(efeat, dst)


def _node_mlp(parts, grid_nfeat, Wn1a, Wn1b, bn1, Wn2, bn2, ln_g, ln_b, blk):
    n, d = grid_nfeat.shape

    def body(parts_ref, grid_ref, w1a_ref, w1b_ref, b1_ref, w2_ref, b2_ref,
             lg_ref, lb_ref, out_ref):
        agg = parts_ref[0] + parts_ref[1]
        grid_blk = grid_ref[...]
        h = lax.dot_general(
            agg, w1a_ref[...], (((1,), (1,)), ((), ())),
            preferred_element_type=jnp.float32)
        h = h + lax.dot_general(
            grid_blk, w1b_ref[...], (((1,), (1,)), ((), ())),
            preferred_element_type=jnp.float32) + b1_ref[...]
        h = _silu(h)
        p = lax.dot_general(
            h, w2_ref[...], (((1,), (1,)), ((), ())),
            preferred_element_type=jnp.float32) + b2_ref[...]
        out_ref[...] = _layer_norm(p, lg_ref[...], lb_ref[...]) + grid_blk

    return pl.pallas_call(
        body,
        grid=(n // blk,),
        in_specs=[
            pl.BlockSpec((NUM_CORES, blk, d), lambda i: (0, i, 0)),
            pl.BlockSpec((blk, d), lambda i: (i, 0)),
            pl.BlockSpec(Wn1a.shape, lambda i: (0, 0)),
            pl.BlockSpec(Wn1b.shape, lambda i: (0, 0)),
            pl.BlockSpec((1, d), lambda i: (0, 0)),
            pl.BlockSpec(Wn2.shape, lambda i: (0, 0)),
            pl.BlockSpec((1, d), lambda i: (0, 0)),
            pl.BlockSpec((1, d), lambda i: (0, 0)),
            pl.BlockSpec((1, d), lambda i: (0, 0)),
        ],
        out_specs=pl.BlockSpec((blk, d), lambda i: (i, 0)),
        out_shape=jax.ShapeDtypeStruct((n, d), jnp.float32),
    )(parts, grid_nfeat, Wn1a, Wn1b, bn1, Wn2, bn2, ln_g, ln_b)


def kernel(m2g_efeat, grid_nfeat, mesh_nfeat, edge_index,
           We, Ws, Wd, b1, W2, b2, ln_e_g, ln_e_b,
           Wn1, bn1, Wn2, bn2, ln_n_g, ln_n_b):
    e, d = m2g_efeat.shape
    n = grid_nfeat.shape[0]
    src = edge_index[0].astype(jnp.int32)
    dst = edge_index[1].astype(jnp.int32)

    row = lambda v: v.reshape(1, -1)

    xs, xd = _node_proj(mesh_nfeat, grid_nfeat, Ws, Wd, row(b1), blk=1000)
    g = _gather_pair(xs, xd, src, dst, chunk=400)
    efeat = _edge_mlp(m2g_efeat, g, We, W2, row(b2), row(ln_e_g),
                      row(ln_e_b), blk=2000)
    parts = _scatter_sum(efeat, dst, n, chunk=200)
    out = _node_mlp(parts, grid_nfeat, Wn1[:, :d], Wn1[:, d:], row(bn1),
                    Wn2, row(bn2), row(ln_n_g), row(ln_n_b), blk=1000)
    return out


# speedup vs baseline: 1.2087x; 1.0720x over previous
import functools

import jax
import jax.numpy as jnp
from jax import lax
from jax.experimental import pallas as pl
from jax.experimental.pallas import tpu as pltpu
from jax.experimental.pallas import tpu_sc as plsc

NUM_CORES = 2
NUM_SUBCORES = 16
NW = NUM_CORES * NUM_SUBCORES


def _pack_bf16_pairs(x):
    k = x.shape[-1] // 2
    a = x[..., :k].astype(jnp.bfloat16).astype(jnp.float32)
    b = x[..., k:].astype(jnp.bfloat16).astype(jnp.float32)
    ua = lax.shift_right_logical(lax.bitcast_convert_type(a, jnp.uint32),
                                 jnp.uint32(16))
    ub = lax.bitcast_convert_type(b, jnp.uint32) & jnp.uint32(0xFFFF0000)
    return lax.bitcast_convert_type(ua | ub, jnp.float32)


def _unpack_bf16_pairs(p):
    u = lax.bitcast_convert_type(p, jnp.uint32)
    a = lax.bitcast_convert_type(lax.shift_left(u, jnp.uint32(16)), jnp.float32)
    b = lax.bitcast_convert_type(u & jnp.uint32(0xFFFF0000), jnp.float32)
    return jnp.concatenate([a, b], axis=-1)


def _silu(x):
    return x * jax.nn.sigmoid(x)


def _layer_norm(x, g, b, eps=1e-5):
    m = jnp.mean(x, axis=-1, keepdims=True)
    v = jnp.var(x, axis=-1, keepdims=True)
    return (x - m) / jnp.sqrt(v + eps) * g + b


def _node_proj(mesh_nfeat, grid_nfeat, Ws, Wd, b1, blk):
    n, d = mesh_nfeat.shape

    def body(mesh_ref, grid_ref, ws_ref, wd_ref, b1_ref, xs_ref, xd_ref):
        xs_ref[...] = _pack_bf16_pairs(lax.dot_general(
            mesh_ref[...], ws_ref[...], (((1,), (1,)), ((), ())),
            preferred_element_type=jnp.float32))
        xd_ref[...] = _pack_bf16_pairs(lax.dot_general(
            grid_ref[...], wd_ref[...], (((1,), (1,)), ((), ())),
            preferred_element_type=jnp.float32) + b1_ref[...])

    return pl.pallas_call(
        body,
        grid=(n // blk,),
        in_specs=[
            pl.BlockSpec((blk, d), lambda i: (i, 0)),
            pl.BlockSpec((blk, d), lambda i: (i, 0)),
            pl.BlockSpec(Ws.shape, lambda i: (0, 0)),
            pl.BlockSpec(Wd.shape, lambda i: (0, 0)),
            pl.BlockSpec((1, d), lambda i: (0, 0)),
        ],
        out_specs=[
            pl.BlockSpec((blk, d // 2), lambda i: (i, 0)),
            pl.BlockSpec((blk, d // 2), lambda i: (i, 0)),
        ],
        out_shape=[
            jax.ShapeDtypeStruct((n, d // 2), jnp.float32),
            jax.ShapeDtypeStruct((n, d // 2), jnp.float32),
        ],
    )(mesh_nfeat, grid_nfeat, Ws, Wd, b1)


def _gather_pair(xs, xd, src, dst, chunk):
    e = src.shape[0]
    d = xs.shape[1]
    per_w = e // NW
    n_chunks = per_w // chunk
    mesh = plsc.VectorSubcoreMesh(
        core_axis_name="c", subcore_axis_name="s",
        num_cores=NUM_CORES, num_subcores=NUM_SUBCORES)

    @functools.partial(
        pl.kernel,
        out_type=jax.ShapeDtypeStruct((e, 2 * d), jnp.float32),
        mesh=mesh,
        compiler_params=pltpu.CompilerParams(use_tc_tiling_on_sc=False),
        scratch_types=[
            pltpu.VMEM((chunk,), jnp.int32),
            pltpu.VMEM((chunk,), jnp.int32),
            pltpu.VMEM((chunk, d), jnp.float32),
            pltpu.VMEM((chunk, d), jnp.float32),
            pltpu.SemaphoreType.DMA,
            pltpu.SemaphoreType.DMA,
        ],
    )
    def k(xs_hbm, xd_hbm, src_hbm, dst_hbm, g_hbm,
          idx_s, idx_d, bufa, bufb, sem1, sem2):
        wid = lax.axis_index("s") * NUM_CORES + lax.axis_index("c")

        def chunk_body(c, carry):
            base = wid * per_w + c * chunk
            pltpu.sync_copy(src_hbm.at[pl.ds(base, chunk)], idx_s)
            pltpu.sync_copy(dst_hbm.at[pl.ds(base, chunk)], idx_d)
            cp1 = pltpu.async_copy(xs_hbm.at[idx_s], bufa, sem1)
            cp2 = pltpu.async_copy(xd_hbm.at[idx_d], bufb, sem2)
            cp1.wait()
            cp2.wait()
            pltpu.sync_copy(bufa, g_hbm.at[pl.ds(base, chunk), pl.ds(0, d)])
            pltpu.sync_copy(bufb, g_hbm.at[pl.ds(base, chunk), pl.ds(d, d)])
            return carry

        lax.fori_loop(0, n_chunks, chunk_body, 0)

    return k(xs, xd, src, dst)


def _edge_mlp(m2g, g, We, W2, b2, ln_g, ln_b, blk):
    e, d = m2g.shape

    def body(m2g_ref, g_ref, we_ref, w2_ref, b2_ref, lg_ref, lb_ref,
             out_ref):
        h = lax.dot_general(
            m2g_ref[...], we_ref[...], (((1,), (1,)), ((), ())),
            preferred_element_type=jnp.float32)
        gblk = g_ref[...]
        h = (h + _unpack_bf16_pairs(gblk[:, :d // 2])
             + _unpack_bf16_pairs(gblk[:, d // 2:]))
        h = _silu(h)
        p = lax.dot_general(
            h, w2_ref[...], (((1,), (1,)), ((), ())),
            preferred_element_type=jnp.float32) + b2_ref[...]
        out_ref[...] = _layer_norm(p, lg_ref[...], lb_ref[...])

    return pl.pallas_call(
        body,
        grid=(e // blk,),
        in_specs=[
            pl.BlockSpec((blk, d), lambda i: (i, 0)),
            pl.BlockSpec((blk, d), lambda i: (i, 0)),
            pl.BlockSpec(We.shape, lambda i: (0, 0)),
            pl.BlockSpec(W2.shape, lambda i: (0, 0)),
            pl.BlockSpec((1, d), lambda i: (0, 0)),
            pl.BlockSpec((1, d), lambda i: (0, 0)),
            pl.BlockSpec((1, d), lambda i: (0, 0)),
        ],
        out_specs=pl.BlockSpec((blk, d), lambda i: (i, 0)),
        out_shape=jax.ShapeDtypeStruct((e, d), jnp.float32),
        compiler_params=pltpu.CompilerParams(
            dimension_semantics=("parallel",)),
    )(m2g, g, We, W2, b2, ln_g, ln_b)


def _scatter_sum(efeat, dst, n_nodes, chunk):
    e, d = efeat.shape
    per_w = e // NW
    n_chunks = per_w // chunk
    base_rows = (n_nodes // NUM_SUBCORES) // 8 * 8
    last_rows = n_nodes - base_rows * (NUM_SUBCORES - 1)
    mesh = plsc.VectorSubcoreMesh(
        core_axis_name="c", subcore_axis_name="s",
        num_cores=NUM_CORES, num_subcores=NUM_SUBCORES)

    @functools.partial(
        pl.kernel,
        out_type=jax.ShapeDtypeStruct((NUM_CORES, n_nodes, d), jnp.float32),
        mesh=mesh,
        scratch_types=[
            pltpu.VMEM((chunk,), jnp.int32),
            pltpu.VMEM((chunk, d), jnp.float32),
            pltpu.VMEM_SHARED((n_nodes, d), jnp.float32),
            pltpu.SemaphoreType.DMA,
        ],
    )
    def k(ef_hbm, dst_hbm, out_hbm, idx, rows, acc, sem):
        cid = lax.axis_index("c")
        sid = lax.axis_index("s")
        wid = sid * NUM_CORES + cid

        def zero_row(r, carry):
            for j in range(d // 16):
                rows[r, pl.ds(j * 16, 16)] = jnp.zeros((16,), jnp.float32)
            return carry

        lax.fori_loop(0, chunk, zero_row, 0)
        done = 0
        while done < base_rows:
            step = min(chunk, base_rows - done)
            pltpu.sync_copy(rows.at[pl.ds(0, step)],
                            acc.at[pl.ds(sid * base_rows + done, step)])
            done += step

        extra = last_rows - base_rows

        @pl.when(sid == NUM_SUBCORES - 1)
        def _zero_tail():
            pltpu.sync_copy(
                rows.at[pl.ds(0, extra)],
                acc.at[pl.ds(base_rows * NUM_SUBCORES, extra)])

        plsc.subcore_barrier()

        def chunk_body(c, carry):
            base = wid * per_w + c * chunk
            pltpu.sync_copy(dst_hbm.at[pl.ds(base, chunk)], idx)
            pltpu.sync_copy(ef_hbm.at[pl.ds(base, chunk)], rows)
            pltpu.sync_copy(rows, acc.at[idx], add=True)
            return carry

        lax.fori_loop(0, n_chunks, chunk_body, 0)
        plsc.subcore_barrier()
        pltpu.sync_copy(acc.at[pl.ds(sid * base_rows, base_rows)],
                        out_hbm.at[cid, pl.ds(sid * base_rows, base_rows)])

        @pl.when(sid == NUM_SUBCORES - 1)
        def _write_tail():
            pltpu.sync_copy(
                acc.at[pl.ds(base_rows * NUM_SUBCORES, extra)],
                out_hbm.at[cid, pl.ds(base_rows * NUM_SUBCORES, extra)])

    return k(efeat, dst)


def _node_mlp(parts, grid_nfeat, Wn1a, Wn1b, bn1, Wn2, bn2, ln_g, ln_b, blk):
    n, d = grid_nfeat.shape

    def body(parts_ref, grid_ref, w1a_ref, w1b_ref, b1_ref, w2_ref, b2_ref,
             lg_ref, lb_ref, out_ref):
        agg = parts_ref[0] + parts_ref[1]
        grid_blk = grid_ref[...]
        h = lax.dot_general(
            agg, w1a_ref[...], (((1,), (1,)), ((), ())),
            preferred_element_type=jnp.float32)
        h = h + lax.dot_general(
            grid_blk, w1b_ref[...], (((1,), (1,)), ((), ())),
            preferred_element_type=jnp.float32) + b1_ref[...]
        h = _silu(h)
        p = lax.dot_general(
            h, w2_ref[...], (((1,), (1,)), ((), ())),
            preferred_element_type=jnp.float32) + b2_ref[...]
        out_ref[...] = _layer_norm(p, lg_ref[...], lb_ref[...]) + grid_blk

    return pl.pallas_call(
        body,
        grid=(n // blk,),
        in_specs=[
            pl.BlockSpec((NUM_CORES, blk, d), lambda i: (0, i, 0)),
            pl.BlockSpec((blk, d), lambda i: (i, 0)),
            pl.BlockSpec(Wn1a.shape, lambda i: (0, 0)),
            pl.BlockSpec(Wn1b.shape, lambda i: (0, 0)),
            pl.BlockSpec((1, d), lambda i: (0, 0)),
            pl.BlockSpec(Wn2.shape, lambda i: (0, 0)),
            pl.BlockSpec((1, d), lambda i: (0, 0)),
            pl.BlockSpec((1, d), lambda i: (0, 0)),
            pl.BlockSpec((1, d), lambda i: (0, 0)),
        ],
        out_specs=pl.BlockSpec((blk, d), lambda i: (i, 0)),
        out_shape=jax.ShapeDtypeStruct((n, d), jnp.float32),
    )(parts, grid_nfeat, Wn1a, Wn1b, bn1, Wn2, bn2, ln_g, ln_b)


def kernel(m2g_efeat, grid_nfeat, mesh_nfeat, edge_index,
           We, Ws, Wd, b1, W2, b2, ln_e_g, ln_e_b,
           Wn1, bn1, Wn2, bn2, ln_n_g, ln_n_b):
    e, d = m2g_efeat.shape
    n = grid_nfeat.shape[0]
    src = edge_index[0].astype(jnp.int32)
    dst = edge_index[1].astype(jnp.int32)

    row = lambda v: v.reshape(1, -1)

    xs, xd = _node_proj(mesh_nfeat, grid_nfeat, Ws, Wd, row(b1), blk=1000)
    g = _gather_pair(xs, xd, src, dst, chunk=400)
    efeat = _edge_mlp(m2g_efeat, g, We, W2, row(b2), row(ln_e_g),
                      row(ln_e_b), blk=4000)
    parts = _scatter_sum(efeat, dst, n, chunk=200)
    out = _node_mlp(parts, grid_nfeat, Wn1[:, :d], Wn1[:, d:], row(bn1),
                    Wn2, row(bn2), row(ln_n_g), row(ln_n_b), blk=1000)
    return out


# speedup vs baseline: 1.2248x; 1.0133x over previous
import functools

import jax
import jax.numpy as jnp
from jax import lax
from jax.experimental import pallas as pl
from jax.experimental.pallas import tpu as pltpu
from jax.experimental.pallas import tpu_sc as plsc

NUM_CORES = 2
NUM_SUBCORES = 16
NW = NUM_CORES * NUM_SUBCORES


def _pack_bf16_pairs(x):
    k = x.shape[-1] // 2
    a = x[..., :k].astype(jnp.bfloat16).astype(jnp.float32)
    b = x[..., k:].astype(jnp.bfloat16).astype(jnp.float32)
    ua = lax.shift_right_logical(lax.bitcast_convert_type(a, jnp.uint32),
                                 jnp.uint32(16))
    ub = lax.bitcast_convert_type(b, jnp.uint32) & jnp.uint32(0xFFFF0000)
    return lax.bitcast_convert_type(ua | ub, jnp.float32)


def _unpack_bf16_pairs(p):
    u = lax.bitcast_convert_type(p, jnp.uint32)
    a = lax.bitcast_convert_type(lax.shift_left(u, jnp.uint32(16)), jnp.float32)
    b = lax.bitcast_convert_type(u & jnp.uint32(0xFFFF0000), jnp.float32)
    return jnp.concatenate([a, b], axis=-1)


def _silu(x):
    return x * jax.nn.sigmoid(x)


def _layer_norm(x, g, b, eps=1e-5):
    m = jnp.mean(x, axis=-1, keepdims=True)
    v = jnp.var(x, axis=-1, keepdims=True)
    return (x - m) / jnp.sqrt(v + eps) * g + b


def _node_proj(mesh_nfeat, grid_nfeat, Ws, Wd, b1, blk):
    n, d = mesh_nfeat.shape

    def body(mesh_ref, grid_ref, ws_ref, wd_ref, b1_ref, xs_ref, xd_ref):
        xs_ref[...] = _pack_bf16_pairs(lax.dot_general(
            mesh_ref[...], ws_ref[...], (((1,), (1,)), ((), ())),
            preferred_element_type=jnp.float32))
        xd_ref[...] = _pack_bf16_pairs(lax.dot_general(
            grid_ref[...], wd_ref[...], (((1,), (1,)), ((), ())),
            preferred_element_type=jnp.float32) + b1_ref[...])

    return pl.pallas_call(
        body,
        grid=(n // blk,),
        in_specs=[
            pl.BlockSpec((blk, d), lambda i: (i, 0)),
            pl.BlockSpec((blk, d), lambda i: (i, 0)),
            pl.BlockSpec(Ws.shape, lambda i: (0, 0)),
            pl.BlockSpec(Wd.shape, lambda i: (0, 0)),
            pl.BlockSpec((1, d), lambda i: (0, 0)),
        ],
        out_specs=[
            pl.BlockSpec((blk, d // 2), lambda i: (i, 0)),
            pl.BlockSpec((blk, d // 2), lambda i: (i, 0)),
        ],
        out_shape=[
            jax.ShapeDtypeStruct((n, d // 2), jnp.float32),
            jax.ShapeDtypeStruct((n, d // 2), jnp.float32),
        ],
    )(mesh_nfeat, grid_nfeat, Ws, Wd, b1)


def _gather_pair(xs, xd, src, dst, chunk):
    e = src.shape[0]
    d = xs.shape[1]
    per_w = e // NW
    n_chunks = per_w // chunk
    mesh = plsc.VectorSubcoreMesh(
        core_axis_name="c", subcore_axis_name="s",
        num_cores=NUM_CORES, num_subcores=NUM_SUBCORES)

    @functools.partial(
        pl.kernel,
        out_type=jax.ShapeDtypeStruct((e, 2 * d), jnp.float32),
        mesh=mesh,
        compiler_params=pltpu.CompilerParams(use_tc_tiling_on_sc=False),
        scratch_types=[
            pltpu.VMEM((chunk,), jnp.int32),
            pltpu.VMEM((chunk,), jnp.int32),
            pltpu.VMEM((chunk, d), jnp.float32),
            pltpu.VMEM((chunk, d), jnp.float32),
            pltpu.SemaphoreType.DMA,
            pltpu.SemaphoreType.DMA,
        ],
    )
    def k(xs_hbm, xd_hbm, src_hbm, dst_hbm, g_hbm,
          idx_s, idx_d, bufa, bufb, sem1, sem2):
        wid = lax.axis_index("s") * NUM_CORES + lax.axis_index("c")

        def chunk_body(c, carry):
            base = wid * per_w + c * chunk
            pltpu.sync_copy(src_hbm.at[pl.ds(base, chunk)], idx_s)
            pltpu.sync_copy(dst_hbm.at[pl.ds(base, chunk)], idx_d)
            cp1 = pltpu.async_copy(xs_hbm.at[idx_s], bufa, sem1)
            cp2 = pltpu.async_copy(xd_hbm.at[idx_d], bufb, sem2)
            cp1.wait()
            cp2.wait()
            pltpu.sync_copy(bufa, g_hbm.at[pl.ds(base, chunk), pl.ds(0, d)])
            pltpu.sync_copy(bufb, g_hbm.at[pl.ds(base, chunk), pl.ds(d, d)])
            return carry

        lax.fori_loop(0, n_chunks, chunk_body, 0)

    return k(xs, xd, src, dst)


def _edge_mlp(m2g, g, We, W2, b2, ln_g, ln_b, blk):
    e, d = m2g.shape

    def body(m2g_ref, g_ref, we_ref, w2_ref, b2_ref, lg_ref, lb_ref,
             out_ref):
        h = lax.dot_general(
            m2g_ref[...], we_ref[...], (((1,), (1,)), ((), ())),
            preferred_element_type=jnp.float32)
        gblk = g_ref[...]
        h = (h + _unpack_bf16_pairs(gblk[:, :d // 2])
             + _unpack_bf16_pairs(gblk[:, d // 2:]))
        h = _silu(h)
        p = lax.dot_general(
            h, w2_ref[...], (((1,), (1,)), ((), ())),
            preferred_element_type=jnp.float32) + b2_ref[...]
        out_ref[...] = _layer_norm(p, lg_ref[...], lb_ref[...])

    return pl.pallas_call(
        body,
        grid=(e // blk,),
        in_specs=[
            pl.BlockSpec((blk, d), lambda i: (i, 0)),
            pl.BlockSpec((blk, d), lambda i: (i, 0)),
            pl.BlockSpec(We.shape, lambda i: (0, 0)),
            pl.BlockSpec(W2.shape, lambda i: (0, 0)),
            pl.BlockSpec((1, d), lambda i: (0, 0)),
            pl.BlockSpec((1, d), lambda i: (0, 0)),
            pl.BlockSpec((1, d), lambda i: (0, 0)),
        ],
        out_specs=pl.BlockSpec((blk, d), lambda i: (i, 0)),
        out_shape=jax.ShapeDtypeStruct((e, d), jnp.float32),
        compiler_params=pltpu.CompilerParams(
            dimension_semantics=("parallel",)),
    )(m2g, g, We, W2, b2, ln_g, ln_b)


def _scatter_sum(efeat, dst, n_nodes, chunk):
    e, d = efeat.shape
    per_w = e // NW
    n_chunks = per_w // chunk
    base_rows = (n_nodes // NUM_SUBCORES) // 8 * 8
    last_rows = n_nodes - base_rows * (NUM_SUBCORES - 1)
    mesh = plsc.VectorSubcoreMesh(
        core_axis_name="c", subcore_axis_name="s",
        num_cores=NUM_CORES, num_subcores=NUM_SUBCORES)

    @functools.partial(
        pl.kernel,
        out_type=jax.ShapeDtypeStruct((NUM_CORES, n_nodes, d), jnp.float32),
        mesh=mesh,
        scratch_types=[
            pltpu.VMEM((chunk,), jnp.int32),
            pltpu.VMEM((chunk, d), jnp.float32),
            pltpu.VMEM_SHARED((n_nodes, d), jnp.float32),
            pltpu.SemaphoreType.DMA,
        ],
    )
    def k(ef_hbm, dst_hbm, out_hbm, idx, rows, acc, sem):
        cid = lax.axis_index("c")
        sid = lax.axis_index("s")
        wid = sid * NUM_CORES + cid

        def zero_row(r, carry):
            for j in range(d // 16):
                rows[r, pl.ds(j * 16, 16)] = jnp.zeros((16,), jnp.float32)
            return carry

        lax.fori_loop(0, chunk, zero_row, 0)
        done = 0
        while done < base_rows:
            step = min(chunk, base_rows - done)
            pltpu.sync_copy(rows.at[pl.ds(0, step)],
                            acc.at[pl.ds(sid * base_rows + done, step)])
            done += step

        extra = last_rows - base_rows

        @pl.when(sid == NUM_SUBCORES - 1)
        def _zero_tail():
            pltpu.sync_copy(
                rows.at[pl.ds(0, extra)],
                acc.at[pl.ds(base_rows * NUM_SUBCORES, extra)])

        plsc.subcore_barrier()

        def chunk_body(c, carry):
            base = wid * per_w + c * chunk
            pltpu.sync_copy(dst_hbm.at[pl.ds(base, chunk)], idx)
            pltpu.sync_copy(ef_hbm.at[pl.ds(base, chunk)], rows)
            pltpu.sync_copy(rows, acc.at[idx], add=True)
            return carry

        lax.fori_loop(0, n_chunks, chunk_body, 0)
        plsc.subcore_barrier()
        pltpu.sync_copy(acc.at[pl.ds(sid * base_rows, base_rows)],
                        out_hbm.at[cid, pl.ds(sid * base_rows, base_rows)])

        @pl.when(sid == NUM_SUBCORES - 1)
        def _write_tail():
            pltpu.sync_copy(
                acc.at[pl.ds(base_rows * NUM_SUBCORES, extra)],
                out_hbm.at[cid, pl.ds(base_rows * NUM_SUBCORES, extra)])

    return k(efeat, dst)


def _node_mlp(parts, grid_nfeat, Wn1a, Wn1b, bn1, Wn2, bn2, ln_g, ln_b, blk):
    n, d = grid_nfeat.shape

    def body(parts_ref, grid_ref, w1a_ref, w1b_ref, b1_ref, w2_ref, b2_ref,
             lg_ref, lb_ref, out_ref):
        agg = parts_ref[0] + parts_ref[1]
        grid_blk = grid_ref[...]
        h = lax.dot_general(
            agg, w1a_ref[...], (((1,), (1,)), ((), ())),
            preferred_element_type=jnp.float32)
        h = h + lax.dot_general(
            grid_blk, w1b_ref[...], (((1,), (1,)), ((), ())),
            preferred_element_type=jnp.float32) + b1_ref[...]
        h = _silu(h)
        p = lax.dot_general(
            h, w2_ref[...], (((1,), (1,)), ((), ())),
            preferred_element_type=jnp.float32) + b2_ref[...]
        out_ref[...] = _layer_norm(p, lg_ref[...], lb_ref[...]) + grid_blk

    return pl.pallas_call(
        body,
        grid=(n // blk,),
        in_specs=[
            pl.BlockSpec((NUM_CORES, blk, d), lambda i: (0, i, 0)),
            pl.BlockSpec((blk, d), lambda i: (i, 0)),
            pl.BlockSpec(Wn1a.shape, lambda i: (0, 0)),
            pl.BlockSpec(Wn1b.shape, lambda i: (0, 0)),
            pl.BlockSpec((1, d), lambda i: (0, 0)),
            pl.BlockSpec(Wn2.shape, lambda i: (0, 0)),
            pl.BlockSpec((1, d), lambda i: (0, 0)),
            pl.BlockSpec((1, d), lambda i: (0, 0)),
            pl.BlockSpec((1, d), lambda i: (0, 0)),
        ],
        out_specs=pl.BlockSpec((blk, d), lambda i: (i, 0)),
        out_shape=jax.ShapeDtypeStruct((n, d), jnp.float32),
    )(parts, grid_nfeat, Wn1a, Wn1b, bn1, Wn2, bn2, ln_g, ln_b)


def kernel(m2g_efeat, grid_nfeat, mesh_nfeat, edge_index,
           We, Ws, Wd, b1, W2, b2, ln_e_g, ln_e_b,
           Wn1, bn1, Wn2, bn2, ln_n_g, ln_n_b):
    e, d = m2g_efeat.shape
    n = grid_nfeat.shape[0]
    src = edge_index[0].astype(jnp.int32)
    dst = edge_index[1].astype(jnp.int32)

    row = lambda v: v.reshape(1, -1)

    xs, xd = _node_proj(mesh_nfeat, grid_nfeat, Ws, Wd, row(b1), blk=1000)
    g = _gather_pair(xs, xd, src, dst, chunk=400)
    efeat = _edge_mlp(m2g_efeat, g, We, W2, row(b2), row(ln_e_g),
                      row(ln_e_b), blk=8000)
    parts = _scatter_sum(efeat, dst, n, chunk=200)
    out = _node_mlp(parts, grid_nfeat, Wn1[:, :d], Wn1[:, d:], row(bn1),
                    Wn2, row(bn2), row(ln_n_g), row(ln_n_b), blk=1000)
    return out


# speedup vs baseline: 1.3282x; 1.0844x over previous
import functools

import jax
import jax.numpy as jnp
from jax import lax
from jax.experimental import pallas as pl
from jax.experimental.pallas import tpu as pltpu
from jax.experimental.pallas import tpu_sc as plsc

NUM_CORES = 2
NUM_SUBCORES = 16
NW = NUM_CORES * NUM_SUBCORES


def _pack_bf16_pairs(x):
    k = x.shape[-1] // 2
    a = x[..., :k].astype(jnp.bfloat16).astype(jnp.float32)
    b = x[..., k:].astype(jnp.bfloat16).astype(jnp.float32)
    ua = lax.shift_right_logical(lax.bitcast_convert_type(a, jnp.uint32),
                                 jnp.uint32(16))
    ub = lax.bitcast_convert_type(b, jnp.uint32) & jnp.uint32(0xFFFF0000)
    return lax.bitcast_convert_type(ua | ub, jnp.float32)


def _unpack_bf16_pairs(p):
    u = lax.bitcast_convert_type(p, jnp.uint32)
    a = lax.bitcast_convert_type(lax.shift_left(u, jnp.uint32(16)), jnp.float32)
    b = lax.bitcast_convert_type(u & jnp.uint32(0xFFFF0000), jnp.float32)
    return jnp.concatenate([a, b], axis=-1)


def _silu(x):
    return x * jax.nn.sigmoid(x)


def _layer_norm(x, g, b, eps=1e-5):
    m = jnp.mean(x, axis=-1, keepdims=True)
    v = jnp.var(x, axis=-1, keepdims=True)
    return (x - m) / jnp.sqrt(v + eps) * g + b


def _node_proj(mesh_nfeat, grid_nfeat, Ws, Wd, b1, blk):
    n, d = mesh_nfeat.shape

    def body(mesh_ref, grid_ref, ws_ref, wd_ref, b1_ref, xs_ref, xd_ref):
        xs_ref[...] = _pack_bf16_pairs(lax.dot_general(
            mesh_ref[...], ws_ref[...], (((1,), (1,)), ((), ())),
            preferred_element_type=jnp.float32))
        xd_ref[...] = _pack_bf16_pairs(lax.dot_general(
            grid_ref[...], wd_ref[...], (((1,), (1,)), ((), ())),
            preferred_element_type=jnp.float32) + b1_ref[...])

    return pl.pallas_call(
        body,
        grid=(n // blk,),
        in_specs=[
            pl.BlockSpec((blk, d), lambda i: (i, 0)),
            pl.BlockSpec((blk, d), lambda i: (i, 0)),
            pl.BlockSpec(Ws.shape, lambda i: (0, 0)),
            pl.BlockSpec(Wd.shape, lambda i: (0, 0)),
            pl.BlockSpec((1, d), lambda i: (0, 0)),
        ],
        out_specs=[
            pl.BlockSpec((blk, d // 2), lambda i: (i, 0)),
            pl.BlockSpec((blk, d // 2), lambda i: (i, 0)),
        ],
        out_shape=[
            jax.ShapeDtypeStruct((n, d // 2), jnp.float32),
            jax.ShapeDtypeStruct((n, d // 2), jnp.float32),
        ],
    )(mesh_nfeat, grid_nfeat, Ws, Wd, b1)


def _gather_pair(xs, xd, src, dst, off, ec, chunk):
    d = xs.shape[1]
    per_w = ec // NW
    n_chunks = per_w // chunk
    mesh = plsc.VectorSubcoreMesh(
        core_axis_name="c", subcore_axis_name="s",
        num_cores=NUM_CORES, num_subcores=NUM_SUBCORES)

    @functools.partial(
        pl.kernel,
        out_type=jax.ShapeDtypeStruct((ec, 2 * d), jnp.float32),
        mesh=mesh,
        compiler_params=pltpu.CompilerParams(use_tc_tiling_on_sc=False),
        scratch_types=[
            pltpu.VMEM((chunk,), jnp.int32),
            pltpu.VMEM((chunk,), jnp.int32),
            pltpu.VMEM((chunk, d), jnp.float32),
            pltpu.VMEM((chunk, d), jnp.float32),
            pltpu.SemaphoreType.DMA,
            pltpu.SemaphoreType.DMA,
        ],
    )
    def k(xs_hbm, xd_hbm, src_hbm, dst_hbm, g_hbm,
          idx_s, idx_d, bufa, bufb, sem1, sem2):
        wid = lax.axis_index("s") * NUM_CORES + lax.axis_index("c")

        def chunk_body(c, carry):
            base = wid * per_w + c * chunk
            pltpu.sync_copy(src_hbm.at[pl.ds(off + base, chunk)], idx_s)
            pltpu.sync_copy(dst_hbm.at[pl.ds(off + base, chunk)], idx_d)
            cp1 = pltpu.async_copy(xs_hbm.at[idx_s], bufa, sem1)
            cp2 = pltpu.async_copy(xd_hbm.at[idx_d], bufb, sem2)
            cp1.wait()
            cp2.wait()
            pltpu.sync_copy(bufa, g_hbm.at[pl.ds(base, chunk), pl.ds(0, d)])
            pltpu.sync_copy(bufb, g_hbm.at[pl.ds(base, chunk), pl.ds(d, d)])
            return carry

        lax.fori_loop(0, n_chunks, chunk_body, 0)

    return k(xs, xd, src, dst)


def _edge_mlp(m2g, g, We, W2, b2, ln_g, ln_b, off_blk, ec, blk):
    d = m2g.shape[1]

    def body(m2g_ref, g_ref, we_ref, w2_ref, b2_ref, lg_ref, lb_ref,
             out_ref):
        h = lax.dot_general(
            m2g_ref[...], we_ref[...], (((1,), (1,)), ((), ())),
            preferred_element_type=jnp.float32)
        gblk = g_ref[...]
        h = (h + _unpack_bf16_pairs(gblk[:, :d // 2])
             + _unpack_bf16_pairs(gblk[:, d // 2:]))
        h = _silu(h)
        p = lax.dot_general(
            h, w2_ref[...], (((1,), (1,)), ((), ())),
            preferred_element_type=jnp.float32) + b2_ref[...]
        out_ref[...] = _layer_norm(p, lg_ref[...], lb_ref[...])

    return pl.pallas_call(
        body,
        grid=(ec // blk,),
        in_specs=[
            pl.BlockSpec((blk, d), lambda i: (off_blk + i, 0)),
            pl.BlockSpec((blk, d), lambda i: (i, 0)),
            pl.BlockSpec(We.shape, lambda i: (0, 0)),
            pl.BlockSpec(W2.shape, lambda i: (0, 0)),
            pl.BlockSpec((1, d), lambda i: (0, 0)),
            pl.BlockSpec((1, d), lambda i: (0, 0)),
            pl.BlockSpec((1, d), lambda i: (0, 0)),
        ],
        out_specs=pl.BlockSpec((blk, d), lambda i: (i, 0)),
        out_shape=jax.ShapeDtypeStruct((ec, d), jnp.float32),
    )(m2g, g, We, W2, b2, ln_g, ln_b)


def _scatter_sum(efeat, dst, off, n_nodes, chunk):
    ec, d = efeat.shape
    per_w = ec // NW
    n_chunks = per_w // chunk
    base_rows = (n_nodes // NUM_SUBCORES) // 8 * 8
    last_rows = n_nodes - base_rows * (NUM_SUBCORES - 1)
    mesh = plsc.VectorSubcoreMesh(
        core_axis_name="c", subcore_axis_name="s",
        num_cores=NUM_CORES, num_subcores=NUM_SUBCORES)

    @functools.partial(
        pl.kernel,
        out_type=jax.ShapeDtypeStruct((NUM_CORES, n_nodes, d), jnp.float32),
        mesh=mesh,
        scratch_types=[
            pltpu.VMEM((chunk,), jnp.int32),
            pltpu.VMEM((chunk, d), jnp.float32),
            pltpu.VMEM_SHARED((n_nodes, d), jnp.float32),
            pltpu.SemaphoreType.DMA,
        ],
    )
    def k(ef_hbm, dst_hbm, out_hbm, idx, rows, acc, sem):
        cid = lax.axis_index("c")
        sid = lax.axis_index("s")
        wid = sid * NUM_CORES + cid

        def zero_row(r, carry):
            for j in range(d // 16):
                rows[r, pl.ds(j * 16, 16)] = jnp.zeros((16,), jnp.float32)
            return carry

        lax.fori_loop(0, chunk, zero_row, 0)
        done = 0
        while done < base_rows:
            step = min(chunk, base_rows - done)
            pltpu.sync_copy(rows.at[pl.ds(0, step)],
                            acc.at[pl.ds(sid * base_rows + done, step)])
            done += step

        extra = last_rows - base_rows

        @pl.when(sid == NUM_SUBCORES - 1)
        def _zero_tail():
            pltpu.sync_copy(
                rows.at[pl.ds(0, extra)],
                acc.at[pl.ds(base_rows * NUM_SUBCORES, extra)])

        plsc.subcore_barrier()

        def chunk_body(c, carry):
            base = wid * per_w + c * chunk
            pltpu.sync_copy(dst_hbm.at[pl.ds(off + base, chunk)], idx)
            pltpu.sync_copy(ef_hbm.at[pl.ds(base, chunk)], rows)
            pltpu.sync_copy(rows, acc.at[idx], add=True)
            return carry

        lax.fori_loop(0, n_chunks, chunk_body, 0)
        plsc.subcore_barrier()
        pltpu.sync_copy(acc.at[pl.ds(sid * base_rows, base_rows)],
                        out_hbm.at[cid, pl.ds(sid * base_rows, base_rows)])

        @pl.when(sid == NUM_SUBCORES - 1)
        def _write_tail():
            pltpu.sync_copy(
                acc.at[pl.ds(base_rows * NUM_SUBCORES, extra)],
                out_hbm.at[cid, pl.ds(base_rows * NUM_SUBCORES, extra)])

    return k(efeat, dst)


def _node_mlp(parts, grid_nfeat, Wn1a, Wn1b, bn1, Wn2, bn2, ln_g, ln_b, blk):
    n, d = grid_nfeat.shape
    n_parts = len(parts)

    def body(*refs):
        parts_refs = refs[:n_parts]
        (grid_ref, w1a_ref, w1b_ref, b1_ref, w2_ref, b2_ref,
         lg_ref, lb_ref, out_ref) = refs[n_parts:]
        agg = parts_refs[0][0] + parts_refs[0][1]
        for pr in parts_refs[1:]:
            agg = agg + pr[0] + pr[1]
        grid_blk = grid_ref[...]
        h = lax.dot_general(
            agg, w1a_ref[...], (((1,), (1,)), ((), ())),
            preferred_element_type=jnp.float32)
        h = h + lax.dot_general(
            grid_blk, w1b_ref[...], (((1,), (1,)), ((), ())),
            preferred_element_type=jnp.float32) + b1_ref[...]
        h = _silu(h)
        p = lax.dot_general(
            h, w2_ref[...], (((1,), (1,)), ((), ())),
            preferred_element_type=jnp.float32) + b2_ref[...]
        out_ref[...] = _layer_norm(p, lg_ref[...], lb_ref[...]) + grid_blk

    return pl.pallas_call(
        body,
        grid=(n // blk,),
        in_specs=[
            pl.BlockSpec((NUM_CORES, blk, d), lambda i: (0, i, 0))
            for _ in range(n_parts)
        ] + [
            pl.BlockSpec((blk, d), lambda i: (i, 0)),
            pl.BlockSpec(Wn1a.shape, lambda i: (0, 0)),
            pl.BlockSpec(Wn1b.shape, lambda i: (0, 0)),
            pl.BlockSpec((1, d), lambda i: (0, 0)),
            pl.BlockSpec(Wn2.shape, lambda i: (0, 0)),
            pl.BlockSpec((1, d), lambda i: (0, 0)),
            pl.BlockSpec((1, d), lambda i: (0, 0)),
            pl.BlockSpec((1, d), lambda i: (0, 0)),
        ],
        out_specs=pl.BlockSpec((blk, d), lambda i: (i, 0)),
        out_shape=jax.ShapeDtypeStruct((n, d), jnp.float32),
    )(*parts, grid_nfeat, Wn1a, Wn1b, bn1, Wn2, bn2, ln_g, ln_b)


def kernel(m2g_efeat, grid_nfeat, mesh_nfeat, edge_index,
           We, Ws, Wd, b1, W2, b2, ln_e_g, ln_e_b,
           Wn1, bn1, Wn2, bn2, ln_n_g, ln_n_b):
    e, d = m2g_efeat.shape
    n = grid_nfeat.shape[0]
    src = edge_index[0].astype(jnp.int32)
    dst = edge_index[1].astype(jnp.int32)

    row = lambda v: v.reshape(1, -1)

    xs, xd = _node_proj(mesh_nfeat, grid_nfeat, Ws, Wd, row(b1), blk=1000)

    K = 5
    ec = e // K
    blk = 8000
    parts = []
    for k in range(K):
        g = _gather_pair(xs, xd, src, dst, off=k * ec, ec=ec, chunk=400)
        efeat = _edge_mlp(m2g_efeat, g, We, W2, row(b2), row(ln_e_g),
                          row(ln_e_b), off_blk=k * (ec // blk), ec=ec, blk=blk)
        parts.append(_scatter_sum(efeat, dst, off=k * ec, n_nodes=n, chunk=200))

    out = _node_mlp(parts, grid_nfeat, Wn1[:, :d], Wn1[:, d:], row(bn1),
                    Wn2, row(bn2), row(ln_n_g), row(ln_n_b), blk=1000)
    return out


# speedup vs baseline: 1.3592x; 1.0234x over previous
import functools

import jax
import jax.numpy as jnp
from jax import lax
from jax.experimental import pallas as pl
from jax.experimental.pallas import tpu as pltpu
from jax.experimental.pallas import tpu_sc as plsc

NUM_CORES = 2
NUM_SUBCORES = 16
NW = NUM_CORES * NUM_SUBCORES


def _pack_bf16_pairs(x):
    k = x.shape[-1] // 2
    a = x[..., :k].astype(jnp.bfloat16).astype(jnp.float32)
    b = x[..., k:].astype(jnp.bfloat16).astype(jnp.float32)
    ua = lax.shift_right_logical(lax.bitcast_convert_type(a, jnp.uint32),
                                 jnp.uint32(16))
    ub = lax.bitcast_convert_type(b, jnp.uint32) & jnp.uint32(0xFFFF0000)
    return lax.bitcast_convert_type(ua | ub, jnp.float32)


def _unpack_bf16_pairs(p):
    u = lax.bitcast_convert_type(p, jnp.uint32)
    a = lax.bitcast_convert_type(lax.shift_left(u, jnp.uint32(16)), jnp.float32)
    b = lax.bitcast_convert_type(u & jnp.uint32(0xFFFF0000), jnp.float32)
    return jnp.concatenate([a, b], axis=-1)


def _silu(x):
    return x * jax.nn.sigmoid(x)


def _layer_norm(x, g, b, eps=1e-5):
    m = jnp.mean(x, axis=-1, keepdims=True)
    v = jnp.var(x, axis=-1, keepdims=True)
    return (x - m) / jnp.sqrt(v + eps) * g + b


def _node_proj(mesh_nfeat, grid_nfeat, Ws, Wd, b1, blk):
    n, d = mesh_nfeat.shape

    def body(mesh_ref, grid_ref, ws_ref, wd_ref, b1_ref, xs_ref, xd_ref):
        xs_ref[...] = _pack_bf16_pairs(lax.dot_general(
            mesh_ref[...], ws_ref[...], (((1,), (1,)), ((), ())),
            preferred_element_type=jnp.float32))
        xd_ref[...] = _pack_bf16_pairs(lax.dot_general(
            grid_ref[...], wd_ref[...], (((1,), (1,)), ((), ())),
            preferred_element_type=jnp.float32) + b1_ref[...])

    return pl.pallas_call(
        body,
        grid=(n // blk,),
        in_specs=[
            pl.BlockSpec((blk, d), lambda i: (i, 0)),
            pl.BlockSpec((blk, d), lambda i: (i, 0)),
            pl.BlockSpec(Ws.shape, lambda i: (0, 0)),
            pl.BlockSpec(Wd.shape, lambda i: (0, 0)),
            pl.BlockSpec((1, d), lambda i: (0, 0)),
        ],
        out_specs=[
            pl.BlockSpec((blk, d // 2), lambda i: (i, 0)),
            pl.BlockSpec((blk, d // 2), lambda i: (i, 0)),
        ],
        out_shape=[
            jax.ShapeDtypeStruct((n, d // 2), jnp.float32),
            jax.ShapeDtypeStruct((n, d // 2), jnp.float32),
        ],
    )(mesh_nfeat, grid_nfeat, Ws, Wd, b1)


def _gather_pair(xs, xd, src, dst, off, ec, chunk):
    d = xs.shape[1]
    per_w = ec // NW
    n_chunks = per_w // chunk
    mesh = plsc.VectorSubcoreMesh(
        core_axis_name="c", subcore_axis_name="s",
        num_cores=NUM_CORES, num_subcores=NUM_SUBCORES)

    @functools.partial(
        pl.kernel,
        out_type=jax.ShapeDtypeStruct((ec, 2 * d), jnp.float32),
        mesh=mesh,
        compiler_params=pltpu.CompilerParams(use_tc_tiling_on_sc=False),
        scratch_types=[
            pltpu.VMEM((chunk,), jnp.int32),
            pltpu.VMEM((chunk,), jnp.int32),
            pltpu.VMEM((chunk,), jnp.int32),
            pltpu.VMEM((chunk,), jnp.int32),
            pltpu.VMEM((chunk, d), jnp.float32),
            pltpu.VMEM((chunk, d), jnp.float32),
            pltpu.VMEM((chunk, d), jnp.float32),
            pltpu.VMEM((chunk, d), jnp.float32),
        ] + [pltpu.SemaphoreType.DMA] * 8,
    )
    def k(xs_hbm, xd_hbm, src_hbm, dst_hbm, g_hbm,
          is0, id0, is1, id1, a0, b0, a1, b1,
          gs0, gd0, gs1, gd1, ws0, wd0, ws1, wd1):
        wid = lax.axis_index("s") * NUM_CORES + lax.axis_index("c")
        idx = [(is0, id0), (is1, id1)]
        buf = [(a0, b0), (a1, b1)]
        gsem = [(gs0, gd0), (gs1, gd1)]
        wsem = [(ws0, wd0), (ws1, wd1)]
        pend_g = [None, None]
        pend_w = [None, None]

        for c in range(n_chunks):
            p = c & 1
            if c > 0:
                q = 1 - p
                for h in pend_g[q]:
                    h.wait()
                base = wid * per_w + (c - 1) * chunk
                pend_w[q] = (
                    pltpu.async_copy(
                        buf[q][0], g_hbm.at[pl.ds(base, chunk), pl.ds(0, d)],
                        wsem[q][0]),
                    pltpu.async_copy(
                        buf[q][1], g_hbm.at[pl.ds(base, chunk), pl.ds(d, d)],
                        wsem[q][1]),
                )
            if pend_w[p] is not None:
                for h in pend_w[p]:
                    h.wait()
                pend_w[p] = None
            base = wid * per_w + c * chunk
            pltpu.sync_copy(src_hbm.at[pl.ds(off + base, chunk)], idx[p][0])
            pltpu.sync_copy(dst_hbm.at[pl.ds(off + base, chunk)], idx[p][1])
            pend_g[p] = (
                pltpu.async_copy(xs_hbm.at[idx[p][0]], buf[p][0], gsem[p][0]),
                pltpu.async_copy(xd_hbm.at[idx[p][1]], buf[p][1], gsem[p][1]),
            )

        p = (n_chunks - 1) & 1
        for h in pend_g[p]:
            h.wait()
        base = wid * per_w + (n_chunks - 1) * chunk
        pltpu.sync_copy(buf[p][0], g_hbm.at[pl.ds(base, chunk), pl.ds(0, d)])
        pltpu.sync_copy(buf[p][1], g_hbm.at[pl.ds(base, chunk), pl.ds(d, d)])
        if pend_w[1 - p] is not None:
            for h in pend_w[1 - p]:
                h.wait()

    return k(xs, xd, src, dst)


def _edge_mlp(m2g, g, We, W2, b2, ln_g, ln_b, off_blk, ec, blk):
    d = m2g.shape[1]

    def body(m2g_ref, g_ref, we_ref, w2_ref, b2_ref, lg_ref, lb_ref,
             out_ref):
        h = lax.dot_general(
            m2g_ref[...], we_ref[...], (((1,), (1,)), ((), ())),
            preferred_element_type=jnp.float32)
        gblk = g_ref[...]
        h = (h + _unpack_bf16_pairs(gblk[:, :d // 2])
             + _unpack_bf16_pairs(gblk[:, d // 2:]))
        h = _silu(h)
        p = lax.dot_general(
            h, w2_ref[...], (((1,), (1,)), ((), ())),
            preferred_element_type=jnp.float32) + b2_ref[...]
        out_ref[...] = _layer_norm(p, lg_ref[...], lb_ref[...])

    return pl.pallas_call(
        body,
        grid=(ec // blk,),
        in_specs=[
            pl.BlockSpec((blk, d), lambda i: (off_blk + i, 0)),
            pl.BlockSpec((blk, d), lambda i: (i, 0)),
            pl.BlockSpec(We.shape, lambda i: (0, 0)),
            pl.BlockSpec(W2.shape, lambda i: (0, 0)),
            pl.BlockSpec((1, d), lambda i: (0, 0)),
            pl.BlockSpec((1, d), lambda i: (0, 0)),
            pl.BlockSpec((1, d), lambda i: (0, 0)),
        ],
        out_specs=pl.BlockSpec((blk, d), lambda i: (i, 0)),
        out_shape=jax.ShapeDtypeStruct((ec, d), jnp.float32),
    )(m2g, g, We, W2, b2, ln_g, ln_b)


def _scatter_sum(efeat, dst, off, n_nodes, chunk):
    ec, d = efeat.shape
    per_w = ec // NW
    n_chunks = per_w // chunk
    base_rows = (n_nodes // NUM_SUBCORES) // 8 * 8
    last_rows = n_nodes - base_rows * (NUM_SUBCORES - 1)
    mesh = plsc.VectorSubcoreMesh(
        core_axis_name="c", subcore_axis_name="s",
        num_cores=NUM_CORES, num_subcores=NUM_SUBCORES)

    @functools.partial(
        pl.kernel,
        out_type=jax.ShapeDtypeStruct((NUM_CORES, n_nodes, d), jnp.float32),
        mesh=mesh,
        scratch_types=[
            pltpu.VMEM((chunk,), jnp.int32),
            pltpu.VMEM((chunk, d), jnp.float32),
            pltpu.VMEM_SHARED((n_nodes, d), jnp.float32),
            pltpu.SemaphoreType.DMA,
        ],
    )
    def k(ef_hbm, dst_hbm, out_hbm, idx, rows, acc, sem):
        cid = lax.axis_index("c")
        sid = lax.axis_index("s")
        wid = sid * NUM_CORES + cid

        def zero_row(r, carry):
            for j in range(d // 16):
                rows[r, pl.ds(j * 16, 16)] = jnp.zeros((16,), jnp.float32)
            return carry

        lax.fori_loop(0, chunk, zero_row, 0)
        done = 0
        while done < base_rows:
            step = min(chunk, base_rows - done)
            pltpu.sync_copy(rows.at[pl.ds(0, step)],
                            acc.at[pl.ds(sid * base_rows + done, step)])
            done += step

        extra = last_rows - base_rows

        @pl.when(sid == NUM_SUBCORES - 1)
        def _zero_tail():
            pltpu.sync_copy(
                rows.at[pl.ds(0, extra)],
                acc.at[pl.ds(base_rows * NUM_SUBCORES, extra)])

        plsc.subcore_barrier()

        def chunk_body(c, carry):
            base = wid * per_w + c * chunk
            pltpu.sync_copy(dst_hbm.at[pl.ds(off + base, chunk)], idx)
            pltpu.sync_copy(ef_hbm.at[pl.ds(base, chunk)], rows)
            pltpu.sync_copy(rows, acc.at[idx], add=True)
            return carry

        lax.fori_loop(0, n_chunks, chunk_body, 0)
        plsc.subcore_barrier()
        pltpu.sync_copy(acc.at[pl.ds(sid * base_rows, base_rows)],
                        out_hbm.at[cid, pl.ds(sid * base_rows, base_rows)])

        @pl.when(sid == NUM_SUBCORES - 1)
        def _write_tail():
            pltpu.sync_copy(
                acc.at[pl.ds(base_rows * NUM_SUBCORES, extra)],
                out_hbm.at[cid, pl.ds(base_rows * NUM_SUBCORES, extra)])

    return k(efeat, dst)


def _node_mlp(parts, grid_nfeat, Wn1a, Wn1b, bn1, Wn2, bn2, ln_g, ln_b, blk):
    n, d = grid_nfeat.shape
    n_parts = len(parts)

    def body(*refs):
        parts_refs = refs[:n_parts]
        (grid_ref, w1a_ref, w1b_ref, b1_ref, w2_ref, b2_ref,
         lg_ref, lb_ref, out_ref) = refs[n_parts:]
        agg = parts_refs[0][0] + parts_refs[0][1]
        for pr in parts_refs[1:]:
            agg = agg + pr[0] + pr[1]
        grid_blk = grid_ref[...]
        h = lax.dot_general(
            agg, w1a_ref[...], (((1,), (1,)), ((), ())),
            preferred_element_type=jnp.float32)
        h = h + lax.dot_general(
            grid_blk, w1b_ref[...], (((1,), (1,)), ((), ())),
            preferred_element_type=jnp.float32) + b1_ref[...]
        h = _silu(h)
        p = lax.dot_general(
            h, w2_ref[...], (((1,), (1,)), ((), ())),
            preferred_element_type=jnp.float32) + b2_ref[...]
        out_ref[...] = _layer_norm(p, lg_ref[...], lb_ref[...]) + grid_blk

    return pl.pallas_call(
        body,
        grid=(n // blk,),
        in_specs=[
            pl.BlockSpec((NUM_CORES, blk, d), lambda i: (0, i, 0))
            for _ in range(n_parts)
        ] + [
            pl.BlockSpec((blk, d), lambda i: (i, 0)),
            pl.BlockSpec(Wn1a.shape, lambda i: (0, 0)),
            pl.BlockSpec(Wn1b.shape, lambda i: (0, 0)),
            pl.BlockSpec((1, d), lambda i: (0, 0)),
            pl.BlockSpec(Wn2.shape, lambda i: (0, 0)),
            pl.BlockSpec((1, d), lambda i: (0, 0)),
            pl.BlockSpec((1, d), lambda i: (0, 0)),
            pl.BlockSpec((1, d), lambda i: (0, 0)),
        ],
        out_specs=pl.BlockSpec((blk, d), lambda i: (i, 0)),
        out_shape=jax.ShapeDtypeStruct((n, d), jnp.float32),
    )(*parts, grid_nfeat, Wn1a, Wn1b, bn1, Wn2, bn2, ln_g, ln_b)


def kernel(m2g_efeat, grid_nfeat, mesh_nfeat, edge_index,
           We, Ws, Wd, b1, W2, b2, ln_e_g, ln_e_b,
           Wn1, bn1, Wn2, bn2, ln_n_g, ln_n_b):
    e, d = m2g_efeat.shape
    n = grid_nfeat.shape[0]
    src = edge_index[0].astype(jnp.int32)
    dst = edge_index[1].astype(jnp.int32)

    row = lambda v: v.reshape(1, -1)

    xs, xd = _node_proj(mesh_nfeat, grid_nfeat, Ws, Wd, row(b1), blk=1000)

    K = 5
    ec = e // K
    blk = 8000
    parts = []
    for k in range(K):
        g = _gather_pair(xs, xd, src, dst, off=k * ec, ec=ec, chunk=400)
        efeat = _edge_mlp(m2g_efeat, g, We, W2, row(b2), row(ln_e_g),
                          row(ln_e_b), off_blk=k * (ec // blk), ec=ec, blk=blk)
        parts.append(_scatter_sum(efeat, dst, off=k * ec, n_nodes=n, chunk=200))

    out = _node_mlp(parts, grid_nfeat, Wn1[:, :d], Wn1[:, d:], row(bn1),
                    Wn2, row(bn2), row(ln_n_g), row(ln_n_b), blk=1000)
    return out


# speedup vs baseline: 1.4276x; 1.0503x over previous
import functools

import jax
import jax.numpy as jnp
from jax import lax
from jax.experimental import pallas as pl
from jax.experimental.pallas import tpu as pltpu
from jax.experimental.pallas import tpu_sc as plsc

NUM_CORES = 2
NUM_SUBCORES = 16
NW = NUM_CORES * NUM_SUBCORES


def _pack_bf16_pairs(x):
    k = x.shape[-1] // 2
    a = x[..., :k].astype(jnp.bfloat16).astype(jnp.float32)
    b = x[..., k:].astype(jnp.bfloat16).astype(jnp.float32)
    ua = lax.shift_right_logical(lax.bitcast_convert_type(a, jnp.uint32),
                                 jnp.uint32(16))
    ub = lax.bitcast_convert_type(b, jnp.uint32) & jnp.uint32(0xFFFF0000)
    return lax.bitcast_convert_type(ua | ub, jnp.float32)


def _unpack_bf16_pairs(p):
    u = lax.bitcast_convert_type(p, jnp.uint32)
    a = lax.bitcast_convert_type(lax.shift_left(u, jnp.uint32(16)), jnp.float32)
    b = lax.bitcast_convert_type(u & jnp.uint32(0xFFFF0000), jnp.float32)
    return jnp.concatenate([a, b], axis=-1)


def _silu(x):
    return x * jax.nn.sigmoid(x)


def _layer_norm(x, g, b, eps=1e-5):
    m = jnp.mean(x, axis=-1, keepdims=True)
    v = jnp.var(x, axis=-1, keepdims=True)
    return (x - m) / jnp.sqrt(v + eps) * g + b


def _node_proj(mesh_nfeat, grid_nfeat, Ws, Wd, b1, blk):
    n, d = mesh_nfeat.shape

    def body(mesh_ref, grid_ref, ws_ref, wd_ref, b1_ref, xs_ref, xd_ref):
        xs_ref[...] = _pack_bf16_pairs(lax.dot_general(
            mesh_ref[...], ws_ref[...], (((1,), (1,)), ((), ())),
            preferred_element_type=jnp.float32))
        xd_ref[...] = _pack_bf16_pairs(lax.dot_general(
            grid_ref[...], wd_ref[...], (((1,), (1,)), ((), ())),
            preferred_element_type=jnp.float32) + b1_ref[...])

    return pl.pallas_call(
        body,
        grid=(n // blk,),
        in_specs=[
            pl.BlockSpec((blk, d), lambda i: (i, 0)),
            pl.BlockSpec((blk, d), lambda i: (i, 0)),
            pl.BlockSpec(Ws.shape, lambda i: (0, 0)),
            pl.BlockSpec(Wd.shape, lambda i: (0, 0)),
            pl.BlockSpec((1, d), lambda i: (0, 0)),
        ],
        out_specs=[
            pl.BlockSpec((blk, d // 2), lambda i: (i, 0)),
            pl.BlockSpec((blk, d // 2), lambda i: (i, 0)),
        ],
        out_shape=[
            jax.ShapeDtypeStruct((n, d // 2), jnp.float32),
            jax.ShapeDtypeStruct((n, d // 2), jnp.float32),
        ],
    )(mesh_nfeat, grid_nfeat, Ws, Wd, b1)


def _gather_pair(xs, xd, src, dst, off, ec, chunk):
    d = xs.shape[1]
    per_w = ec // NW
    n_chunks = per_w // chunk
    mesh = plsc.VectorSubcoreMesh(
        core_axis_name="c", subcore_axis_name="s",
        num_cores=NUM_CORES, num_subcores=NUM_SUBCORES)

    @functools.partial(
        pl.kernel,
        out_type=jax.ShapeDtypeStruct((ec, 2 * d), jnp.float32),
        mesh=mesh,
        compiler_params=pltpu.CompilerParams(use_tc_tiling_on_sc=False),
        scratch_types=[
            pltpu.VMEM((chunk,), jnp.int32),
            pltpu.VMEM((chunk,), jnp.int32),
            pltpu.VMEM((chunk,), jnp.int32),
            pltpu.VMEM((chunk,), jnp.int32),
            pltpu.VMEM((chunk, d), jnp.float32),
            pltpu.VMEM((chunk, d), jnp.float32),
            pltpu.VMEM((chunk, d), jnp.float32),
            pltpu.VMEM((chunk, d), jnp.float32),
        ] + [pltpu.SemaphoreType.DMA] * 8,
    )
    def k(xs_hbm, xd_hbm, src_hbm, dst_hbm, g_hbm,
          is0, id0, is1, id1, a0, b0, a1, b1,
          gs0, gd0, gs1, gd1, ws0, wd0, ws1, wd1):
        wid = lax.axis_index("s") * NUM_CORES + lax.axis_index("c")
        idx = [(is0, id0), (is1, id1)]
        buf = [(a0, b0), (a1, b1)]
        gsem = [(gs0, gd0), (gs1, gd1)]
        wsem = [(ws0, wd0), (ws1, wd1)]
        pend_g = [None, None]
        pend_w = [None, None]

        for c in range(n_chunks):
            p = c & 1
            if c > 0:
                q = 1 - p
                for h in pend_g[q]:
                    h.wait()
                base = wid * per_w + (c - 1) * chunk
                pend_w[q] = (
                    pltpu.async_copy(
                        buf[q][0], g_hbm.at[pl.ds(base, chunk), pl.ds(0, d)],
                        wsem[q][0]),
                    pltpu.async_copy(
                        buf[q][1], g_hbm.at[pl.ds(base, chunk), pl.ds(d, d)],
                        wsem[q][1]),
                )
            if pend_w[p] is not None:
                for h in pend_w[p]:
                    h.wait()
                pend_w[p] = None
            base = wid * per_w + c * chunk
            pltpu.sync_copy(src_hbm.at[pl.ds(off + base, chunk)], idx[p][0])
            pltpu.sync_copy(dst_hbm.at[pl.ds(off + base, chunk)], idx[p][1])
            pend_g[p] = (
                pltpu.async_copy(xs_hbm.at[idx[p][0]], buf[p][0], gsem[p][0]),
                pltpu.async_copy(xd_hbm.at[idx[p][1]], buf[p][1], gsem[p][1]),
            )

        p = (n_chunks - 1) & 1
        for h in pend_g[p]:
            h.wait()
        base = wid * per_w + (n_chunks - 1) * chunk
        pltpu.sync_copy(buf[p][0], g_hbm.at[pl.ds(base, chunk), pl.ds(0, d)])
        pltpu.sync_copy(buf[p][1], g_hbm.at[pl.ds(base, chunk), pl.ds(d, d)])
        if pend_w[1 - p] is not None:
            for h in pend_w[1 - p]:
                h.wait()

    return k(xs, xd, src, dst)


def _edge_mlp(m2g, g, We, W2, b2, ln_g, ln_b, off_blk, ec, blk):
    d = m2g.shape[1]

    def body(m2g_ref, g_ref, we_ref, w2_ref, b2_ref, lg_ref, lb_ref,
             out_ref):
        h = lax.dot_general(
            m2g_ref[...], we_ref[...], (((1,), (1,)), ((), ())),
            preferred_element_type=jnp.float32)
        gblk = g_ref[...]
        h = (h + _unpack_bf16_pairs(gblk[:, :d // 2])
             + _unpack_bf16_pairs(gblk[:, d // 2:]))
        h = _silu(h)
        p = lax.dot_general(
            h, w2_ref[...], (((1,), (1,)), ((), ())),
            preferred_element_type=jnp.float32) + b2_ref[...]
        out_ref[...] = _layer_norm(p, lg_ref[...], lb_ref[...])

    return pl.pallas_call(
        body,
        grid=(ec // blk,),
        in_specs=[
            pl.BlockSpec((blk, d), lambda i: (off_blk + i, 0)),
            pl.BlockSpec((blk, d), lambda i: (i, 0)),
            pl.BlockSpec(We.shape, lambda i: (0, 0)),
            pl.BlockSpec(W2.shape, lambda i: (0, 0)),
            pl.BlockSpec((1, d), lambda i: (0, 0)),
            pl.BlockSpec((1, d), lambda i: (0, 0)),
            pl.BlockSpec((1, d), lambda i: (0, 0)),
        ],
        out_specs=pl.BlockSpec((blk, d), lambda i: (i, 0)),
        out_shape=jax.ShapeDtypeStruct((ec, d), jnp.float32),
    )(m2g, g, We, W2, b2, ln_g, ln_b)


def _scatter_sum(efeat, dst, off, n_nodes, chunk):
    ec, d = efeat.shape
    per_w = ec // NW
    n_chunks = per_w // chunk
    base_rows = (n_nodes // NUM_SUBCORES) // 8 * 8
    last_rows = n_nodes - base_rows * (NUM_SUBCORES - 1)
    mesh = plsc.VectorSubcoreMesh(
        core_axis_name="c", subcore_axis_name="s",
        num_cores=NUM_CORES, num_subcores=NUM_SUBCORES)

    @functools.partial(
        pl.kernel,
        out_type=jax.ShapeDtypeStruct((NUM_CORES, n_nodes, d), jnp.float32),
        mesh=mesh,
        scratch_types=[
            pltpu.VMEM((chunk,), jnp.int32),
            pltpu.VMEM((chunk,), jnp.int32),
            pltpu.VMEM((chunk, d), jnp.float32),
            pltpu.VMEM((chunk, d), jnp.float32),
            pltpu.VMEM_SHARED((n_nodes, d), jnp.float32),
            pltpu.SemaphoreType.DMA,
            pltpu.SemaphoreType.DMA,
        ],
    )
    def k(ef_hbm, dst_hbm, out_hbm, i0, i1, r0, r1, acc, s0, s1):
        cid = lax.axis_index("c")
        sid = lax.axis_index("s")
        wid = sid * NUM_CORES + cid
        idx = [i0, i1]
        rows = [r0, r1]
        sems = [s0, s1]

        def zero_row(r, carry):
            for j in range(d // 16):
                r0[r, pl.ds(j * 16, 16)] = jnp.zeros((16,), jnp.float32)
            return carry

        lax.fori_loop(0, chunk, zero_row, 0)
        done = 0
        while done < base_rows:
            step = min(chunk, base_rows - done)
            pltpu.sync_copy(r0.at[pl.ds(0, step)],
                            acc.at[pl.ds(sid * base_rows + done, step)])
            done += step

        extra = last_rows - base_rows

        @pl.when(sid == NUM_SUBCORES - 1)
        def _zero_tail():
            pltpu.sync_copy(
                r0.at[pl.ds(0, extra)],
                acc.at[pl.ds(base_rows * NUM_SUBCORES, extra)])

        plsc.subcore_barrier()

        pend = [None, None]
        for c in range(n_chunks):
            p = c & 1
            base = wid * per_w + c * chunk
            pltpu.sync_copy(dst_hbm.at[pl.ds(off + base, chunk)], idx[p])
            pend[p] = pltpu.async_copy(ef_hbm.at[pl.ds(base, chunk)],
                                       rows[p], sems[p])
            if c > 0:
                q = 1 - p
                pend[q].wait()
                pltpu.sync_copy(rows[q], acc.at[idx[q]], add=True)
        p = (n_chunks - 1) & 1
        pend[p].wait()
        pltpu.sync_copy(rows[p], acc.at[idx[p]], add=True)

        plsc.subcore_barrier()
        pltpu.sync_copy(acc.at[pl.ds(sid * base_rows, base_rows)],
                        out_hbm.at[cid, pl.ds(sid * base_rows, base_rows)])

        @pl.when(sid == NUM_SUBCORES - 1)
        def _write_tail():
            pltpu.sync_copy(
                acc.at[pl.ds(base_rows * NUM_SUBCORES, extra)],
                out_hbm.at[cid, pl.ds(base_rows * NUM_SUBCORES, extra)])

    return k(efeat, dst)


def _node_mlp(parts, grid_nfeat, Wn1a, Wn1b, bn1, Wn2, bn2, ln_g, ln_b, blk):
    n, d = grid_nfeat.shape
    n_parts = len(parts)

    def body(*refs):
        parts_refs = refs[:n_parts]
        (grid_ref, w1a_ref, w1b_ref, b1_ref, w2_ref, b2_ref,
         lg_ref, lb_ref, out_ref) = refs[n_parts:]
        agg = parts_refs[0][0] + parts_refs[0][1]
        for pr in parts_refs[1:]:
            agg = agg + pr[0] + pr[1]
        grid_blk = grid_ref[...]
        h = lax.dot_general(
            agg, w1a_ref[...], (((1,), (1,)), ((), ())),
            preferred_element_type=jnp.float32)
        h = h + lax.dot_general(
            grid_blk, w1b_ref[...], (((1,), (1,)), ((), ())),
            preferred_element_type=jnp.float32) + b1_ref[...]
        h = _silu(h)
        p = lax.dot_general(
            h, w2_ref[...], (((1,), (1,)), ((), ())),
            preferred_element_type=jnp.float32) + b2_ref[...]
        out_ref[...] = _layer_norm(p, lg_ref[...], lb_ref[...]) + grid_blk

    return pl.pallas_call(
        body,
        grid=(n // blk,),
        in_specs=[
            pl.BlockSpec((NUM_CORES, blk, d), lambda i: (0, i, 0))
            for _ in range(n_parts)
        ] + [
            pl.BlockSpec((blk, d), lambda i: (i, 0)),
            pl.BlockSpec(Wn1a.shape, lambda i: (0, 0)),
            pl.BlockSpec(Wn1b.shape, lambda i: (0, 0)),
            pl.BlockSpec((1, d), lambda i: (0, 0)),
            pl.BlockSpec(Wn2.shape, lambda i: (0, 0)),
            pl.BlockSpec((1, d), lambda i: (0, 0)),
            pl.BlockSpec((1, d), lambda i: (0, 0)),
            pl.BlockSpec((1, d), lambda i: (0, 0)),
        ],
        out_specs=pl.BlockSpec((blk, d), lambda i: (i, 0)),
        out_shape=jax.ShapeDtypeStruct((n, d), jnp.float32),
    )(*parts, grid_nfeat, Wn1a, Wn1b, bn1, Wn2, bn2, ln_g, ln_b)


def kernel(m2g_efeat, grid_nfeat, mesh_nfeat, edge_index,
           We, Ws, Wd, b1, W2, b2, ln_e_g, ln_e_b,
           Wn1, bn1, Wn2, bn2, ln_n_g, ln_n_b):
    e, d = m2g_efeat.shape
    n = grid_nfeat.shape[0]
    src = edge_index[0].astype(jnp.int32)
    dst = edge_index[1].astype(jnp.int32)

    row = lambda v: v.reshape(1, -1)

    xs, xd = _node_proj(mesh_nfeat, grid_nfeat, Ws, Wd, row(b1), blk=1000)

    K = 5
    ec = e // K
    blk = 8000
    parts = []
    for k in range(K):
        g = _gather_pair(xs, xd, src, dst, off=k * ec, ec=ec, chunk=400)
        efeat = _edge_mlp(m2g_efeat, g, We, W2, row(b2), row(ln_e_g),
                          row(ln_e_b), off_blk=k * (ec // blk), ec=ec, blk=blk)
        parts.append(_scatter_sum(efeat, dst, off=k * ec, n_nodes=n, chunk=80))

    out = _node_mlp(parts, grid_nfeat, Wn1[:, :d], Wn1[:, d:], row(bn1),
                    Wn2, row(bn2), row(ln_n_g), row(ln_n_b), blk=1000)
    return out
